# baseline (speedup 1.0000x reference)
import functools

import jax
import jax.numpy as jnp
from jax import lax
from jax.experimental import pallas as pl
from jax.experimental.pallas import tpu as pltpu
from jax.experimental.pallas import tpu_sc as plsc

N = 10000
D = 128
E = 320000
NC = 2
NS = 16
NW = NC * NS
EPW = E // NW
WIN = 40
NWIN = EPW // WIN
AWIN = 128
APW = 10240
ANW = APW // AWIN
NPADW = APW - EPW
PADK = NW * NPADW // 16
RPW = N // NS
NPAD = 10240
DCH = NPAD // NS

_mesh = plsc.VectorSubcoreMesh(core_axis_name="c", subcore_axis_name="s")


def _deg_body(dst_hbm, ones_hbm, zeros_hbm, out_hbm, idx_v, ones_v, deg_sh, sem):
    c = lax.axis_index("c")
    s = lax.axis_index("s")
    wid = s * NC + c
    pltpu.sync_copy(zeros_hbm.at[pl.ds(s * DCH, DCH)],
                    deg_sh.at[pl.ds(s * DCH, DCH)])
    pltpu.sync_copy(dst_hbm.at[wid], idx_v)
    pltpu.sync_copy(ones_hbm, ones_v)
    plsc.subcore_barrier()

    def fire(j, carry):
        pltpu.async_copy(ones_v, deg_sh.at[idx_v.at[j]], sem, add=True)
        return carry

    def drain(j, carry):
        pltpu.make_async_copy(ones_v, deg_sh.at[idx_v.at[j]], sem).wait()
        return carry

    lax.fori_loop(0, ANW, fire, 0)
    lax.fori_loop(0, ANW, drain, 0)
    plsc.subcore_barrier()
    pltpu.sync_copy(deg_sh.at[pl.ds(s * DCH, DCH)],
                    out_hbm.at[c].at[pl.ds(s * DCH, DCH)])


_deg_kernel = functools.partial(
    pl.kernel,
    out_type=jax.ShapeDtypeStruct((NC, NPAD), jnp.float32),
    mesh=_mesh,
    scratch_types=[
        pltpu.VMEM((ANW, AWIN), jnp.int32),
        pltpu.VMEM((AWIN,), jnp.float32),
        pltpu.VMEM_SHARED((NPAD,), jnp.float32),
        pltpu.SemaphoreType.DMA,
    ],
)(_deg_body)


def _agg_body(y_hbm, src_hbm, dst_hbm, zeros_hbm, out_hbm,
              srcv, dstg, rows_a, rows_b, acc_sh,
              sem_ga, sem_gb, sem_sa, sem_sb):
    c = lax.axis_index("c")
    s = lax.axis_index("s")
    wid = s * NC + c
    pltpu.sync_copy(zeros_hbm.at[s], acc_sh.at[pl.ds(s * RPW, RPW)])
    pltpu.sync_copy(src_hbm.at[wid], srcv)
    plsc.subcore_barrier()

    def g_start(w, buf, sem):
        pltpu.async_copy(y_hbm.at[srcv.at[pl.ds(w * AWIN, AWIN)]], buf, sem)

    def g_wait(w, buf, sem):
        pltpu.make_async_copy(y_hbm.at[srcv.at[pl.ds(w * AWIN, AWIN)]],
                              buf, sem).wait()

    def s_start(w, buf, sem):
        pltpu.async_copy(buf, acc_sh.at[dstg.at[w % 8]], sem, add=True)

    def s_wait(w, buf, sem):
        pltpu.make_async_copy(buf, acc_sh.at[dstg.at[w % 8]], sem).wait()

    def i_load(grp):
        off = pl.multiple_of(grp * 8, 8)
        pltpu.sync_copy(dst_hbm.at[wid].at[pl.ds(off, 8)], dstg)

    i_load(0)
    g_start(0, rows_a, sem_ga)

    def step(j, carry):
        w = 2 * j
        g_wait(w, rows_a, sem_ga)

        @pl.when(j > 0)
        def _():
            s_wait(w - 1, rows_b, sem_sb)

        @pl.when(jnp.logical_and(j % 4 == 0, j > 0))
        def _():
            i_load(j // 4)

        g_start(w + 1, rows_b, sem_gb)
        s_start(w, rows_a, sem_sa)
        g_wait(w + 1, rows_b, sem_gb)
        s_wait(w, rows_a, sem_sa)

        @pl.when(j < ANW // 2 - 1)
        def _():
            g_start(w + 2, rows_a, sem_ga)

        s_start(w + 1, rows_b, sem_sb)
        return carry

    lax.fori_loop(0, ANW // 2, step, 0)
    s_wait(ANW - 1, rows_b, sem_sb)
    plsc.subcore_barrier()
    pltpu.sync_copy(acc_sh.at[pl.ds(s * RPW, RPW)], out_hbm.at[c].at[s])


_agg_kernel = functools.partial(
    pl.kernel,
    out_type=jax.ShapeDtypeStruct((NC, NS, RPW, D), jnp.float32),
    mesh=_mesh,
    scratch_types=[
        pltpu.VMEM((APW,), jnp.int32),
        pltpu.VMEM((8, AWIN), jnp.int32),
        pltpu.VMEM((AWIN, D), jnp.float32),
        pltpu.VMEM((AWIN, D), jnp.float32),
        pltpu.VMEM_SHARED((N, D), jnp.float32),
        pltpu.SemaphoreType.DMA,
        pltpu.SemaphoreType.DMA,
        pltpu.SemaphoreType.DMA,
        pltpu.SemaphoreType.DMA,
    ],
)(_agg_body)


def _matmul_body(x_ref, w_ref, xw_ref):
    xw_ref[...] = jnp.dot(x_ref[...], w_ref[...],
                          preferred_element_type=jnp.float32)


_TCB = 1000


def _matmul(x, weight):
    return pl.pallas_call(
        _matmul_body,
        grid=(N // _TCB,),
        in_specs=[
            pl.BlockSpec((_TCB, D), lambda i: (i, 0)),
            pl.BlockSpec((D, D), lambda i: (0, 0)),
        ],
        out_specs=pl.BlockSpec((_TCB, D), lambda i: (i, 0)),
        out_shape=jax.ShapeDtypeStruct((N, D), jnp.float32),
    )(x, weight)


def _scale_body(degp_ref, x_ref, y_ref):
    row = (lax.broadcasted_iota(jnp.int32, (_TCB, 1), 0)
           + pl.program_id(0) * _TCB)
    padc = jnp.where(row < 16, float(PADK), 0.0)
    deg = degp_ref[0] + degp_ref[1] + 1.0 - padc
    dinv = lax.rsqrt(deg)
    y_ref[...] = x_ref[...] * dinv


def _scale(deg_partials, x):
    return pl.pallas_call(
        _scale_body,
        grid=(N // _TCB,),
        in_specs=[
            pl.BlockSpec((2, _TCB, 1), lambda i: (0, i, 0)),
            pl.BlockSpec((_TCB, D), lambda i: (i, 0)),
        ],
        out_specs=pl.BlockSpec((_TCB, D), lambda i: (i, 0)),
        out_shape=jax.ShapeDtypeStruct((N, D), jnp.float32),
    )(deg_partials, x)


def _out_body(p_ref, y_ref, degp_ref, b_ref, o_ref):
    row = (lax.broadcasted_iota(jnp.int32, (_TCB, 1), 0)
           + pl.program_id(0) * _TCB)
    padc = jnp.where(row < 16, float(PADK), 0.0)
    deg = degp_ref[0] + degp_ref[1] + 1.0 - padc
    dinv = lax.rsqrt(deg)
    yfac = jnp.where(row < 16, 1.0 - PADK, 1.0)
    fr = (p_ref[0] + p_ref[1] + yfac * y_ref[...]) * dinv
    o_ref[...] = fr + b_ref[...]


def _combine(partials, y, deg_partials, bias):
    return pl.pallas_call(
        _out_body,
        grid=(N // _TCB,),
        in_specs=[
            pl.BlockSpec((2, _TCB, D), lambda i: (0, i, 0)),
            pl.BlockSpec((_TCB, D), lambda i: (i, 0)),
            pl.BlockSpec((2, _TCB, 1), lambda i: (0, i, 0)),
            pl.BlockSpec((1, D), lambda i: (0, 0)),
        ],
        out_specs=pl.BlockSpec((_TCB, D), lambda i: (i, 0)),
        out_shape=jax.ShapeDtypeStruct((N, D), jnp.float32),
    )(partials, y, deg_partials, bias.reshape(1, D))


def kernel(x, edge_index, W1, b1, W2, b2, weight, bias):
    pad = jnp.tile(jnp.arange(16, dtype=edge_index.dtype), (NW, NPADW // 16))
    srcp = jnp.concatenate([edge_index[0].reshape(NW, EPW), pad], axis=1)
    dstp = jnp.concatenate([edge_index[1].reshape(NW, EPW), pad], axis=1)
    dstp = dstp.reshape(NW, ANW, AWIN)
    ones_w = jnp.ones((AWIN,), jnp.float32)
    zeros_deg = jnp.zeros((NPAD,), jnp.float32)
    zeros_acc = jnp.zeros((NS, RPW, D), jnp.float32)

    deg_partials = _deg_kernel(dstp, ones_w, zeros_deg)[:, :N, None]
    xw = _matmul(x, weight)
    y = _scale(deg_partials, xw)
    partials = _agg_kernel(y, srcp, dstp, zeros_acc).reshape(NC, N, D)
    return _combine(partials, y, deg_partials, bias)


# speedup vs baseline: 1.0350x; 1.0350x over previous
import functools

import jax
import jax.numpy as jnp
from jax import lax
from jax.experimental import pallas as pl
from jax.experimental.pallas import tpu as pltpu
from jax.experimental.pallas import tpu_sc as plsc

N = 10000
D = 128
E = 320000
NC = 2
NS = 16
NW = NC * NS
EPW = E // NW
WIN = 40
NWIN = EPW // WIN
AWIN = 128
APW = 10240
ANW = APW // AWIN
NPADW = APW - EPW
PADK = NW * NPADW // 16
RPW = N // NS
NPAD = 10240
DCH = NPAD // NS

_mesh = plsc.VectorSubcoreMesh(core_axis_name="c", subcore_axis_name="s")


def _deg_body(dst_hbm, ones_hbm, zeros_hbm, out_hbm, idx_v, ones_v, deg_sh, sem):
    c = lax.axis_index("c")
    s = lax.axis_index("s")
    wid = s * NC + c
    pltpu.sync_copy(zeros_hbm.at[pl.ds(s * DCH, DCH)],
                    deg_sh.at[pl.ds(s * DCH, DCH)])
    pltpu.sync_copy(dst_hbm.at[wid], idx_v)
    pltpu.sync_copy(ones_hbm, ones_v)
    plsc.subcore_barrier()

    def fire(j, carry):
        pltpu.async_copy(ones_v, deg_sh.at[idx_v.at[j]], sem, add=True)
        return carry

    def drain(j, carry):
        pltpu.make_async_copy(ones_v, deg_sh.at[idx_v.at[j]], sem).wait()
        return carry

    lax.fori_loop(0, ANW, fire, 0)
    lax.fori_loop(0, ANW, drain, 0)
    plsc.subcore_barrier()
    pltpu.sync_copy(deg_sh.at[pl.ds(s * DCH, DCH)],
                    out_hbm.at[c].at[pl.ds(s * DCH, DCH)])


_deg_kernel = functools.partial(
    pl.kernel,
    out_type=jax.ShapeDtypeStruct((NC, NPAD), jnp.float32),
    mesh=_mesh,
    scratch_types=[
        pltpu.VMEM((ANW, AWIN), jnp.int32),
        pltpu.VMEM((AWIN,), jnp.float32),
        pltpu.VMEM_SHARED((NPAD,), jnp.float32),
        pltpu.SemaphoreType.DMA,
    ],
)(_deg_body)


def _agg_body(y_hbm, src_hbm, dst_hbm, zeros_hbm, out_hbm,
              srcv, dstg, rows_a, rows_b, acc_sh,
              sem_ga, sem_gb, sem_sa, sem_sb):
    c = lax.axis_index("c")
    s = lax.axis_index("s")
    wid = s * NC + c
    pltpu.sync_copy(zeros_hbm.at[s], acc_sh.at[pl.ds(s * RPW, RPW)])
    pltpu.sync_copy(src_hbm.at[wid], srcv)
    plsc.subcore_barrier()

    rows = (rows_a, rows_b)
    sg = (sem_ga, sem_gb)
    ss = (sem_sa, sem_sb)
    NB = ANW // 8

    def g_start(w, k):
        pltpu.async_copy(y_hbm.at[srcv.at[pl.ds(w * AWIN, AWIN)]],
                         rows[k % 2], sg[k % 2])

    def g_wait(w, k):
        pltpu.make_async_copy(y_hbm.at[srcv.at[pl.ds(w * AWIN, AWIN)]],
                              rows[k % 2], sg[k % 2]).wait()

    def s_start(k):
        pltpu.async_copy(rows[k % 2], acc_sh.at[dstg.at[k % 8]],
                         ss[k % 2], add=True)

    def s_wait(k):
        pltpu.make_async_copy(rows[k % 2], acc_sh.at[dstg.at[k % 8]],
                              ss[k % 2]).wait()

    def i_load(j):
        off = pl.multiple_of(j * 8, 8)
        pltpu.sync_copy(dst_hbm.at[wid].at[pl.ds(off, 8)], dstg)

    g_start(0, 0)

    def step(j, carry):
        w0 = 8 * j

        @pl.when(j > 0)
        def _():
            s_wait(7)

        i_load(j)
        for k in range(8):
            w = w0 + k
            g_wait(w, k)
            if k > 0:
                s_wait(k - 1)
            if k == 7:
                @pl.when(j < NB - 1)
                def _():
                    g_start(w + 1, k + 1)
            else:
                g_start(w + 1, k + 1)
            s_start(k)
        return carry

    lax.fori_loop(0, NB, step, 0)
    s_wait(7)
    plsc.subcore_barrier()
    pltpu.sync_copy(acc_sh.at[pl.ds(s * RPW, RPW)], out_hbm.at[c].at[s])


_agg_kernel = functools.partial(
    pl.kernel,
    out_type=jax.ShapeDtypeStruct((NC, NS, RPW, D), jnp.float32),
    mesh=_mesh,
    scratch_types=[
        pltpu.VMEM((APW,), jnp.int32),
        pltpu.VMEM((8, AWIN), jnp.int32),
        pltpu.VMEM((AWIN, D), jnp.float32),
        pltpu.VMEM((AWIN, D), jnp.float32),
        pltpu.VMEM_SHARED((N, D), jnp.float32),
        pltpu.SemaphoreType.DMA,
        pltpu.SemaphoreType.DMA,
        pltpu.SemaphoreType.DMA,
        pltpu.SemaphoreType.DMA,
    ],
)(_agg_body)


def _matmul_body(x_ref, w_ref, xw_ref):
    xw_ref[...] = jnp.dot(x_ref[...], w_ref[...],
                          preferred_element_type=jnp.float32)


def _matmul(x, weight):
    return pl.pallas_call(
        _matmul_body,
        out_shape=jax.ShapeDtypeStruct((N, D), jnp.float32),
    )(x, weight)


def _scale_body(degp_ref, x_ref, y_ref):
    row = lax.broadcasted_iota(jnp.int32, (N, 1), 0)
    padc = jnp.where(row < 16, float(PADK), 0.0)
    deg = degp_ref[0] + degp_ref[1] + 1.0 - padc
    dinv = lax.rsqrt(deg)
    y_ref[...] = x_ref[...] * dinv


def _scale(deg_partials, x):
    return pl.pallas_call(
        _scale_body,
        out_shape=jax.ShapeDtypeStruct((N, D), jnp.float32),
    )(deg_partials, x)


def _out_body(p_ref, y_ref, degp_ref, b_ref, o_ref):
    row = lax.broadcasted_iota(jnp.int32, (N, 1), 0)
    padc = jnp.where(row < 16, float(PADK), 0.0)
    deg = degp_ref[0] + degp_ref[1] + 1.0 - padc
    dinv = lax.rsqrt(deg)
    yfac = jnp.where(row < 16, 1.0 - PADK, 1.0)
    fr = (p_ref[0] + p_ref[1] + yfac * y_ref[...]) * dinv
    o_ref[...] = fr + b_ref[...]


def _combine(partials, y, deg_partials, bias):
    return pl.pallas_call(
        _out_body,
        out_shape=jax.ShapeDtypeStruct((N, D), jnp.float32),
    )(partials, y, deg_partials, bias.reshape(1, D))


def kernel(x, edge_index, W1, b1, W2, b2, weight, bias):
    pad = jnp.tile(jnp.arange(16, dtype=edge_index.dtype), (NW, NPADW // 16))
    srcp = jnp.concatenate([edge_index[0].reshape(NW, EPW), pad], axis=1)
    dstp = jnp.concatenate([edge_index[1].reshape(NW, EPW), pad], axis=1)
    dstp = dstp.reshape(NW, ANW, AWIN)
    ones_w = jnp.ones((AWIN,), jnp.float32)
    zeros_deg = jnp.zeros((NPAD,), jnp.float32)
    zeros_acc = jnp.zeros((NS, RPW, D), jnp.float32)

    deg_partials = _deg_kernel(dstp, ones_w, zeros_deg)[:, :N, None]
    xw = _matmul(x, weight)
    y = _scale(deg_partials, xw)
    partials = _agg_kernel(y, srcp, dstp, zeros_acc).reshape(NC, N, D)
    return _combine(partials, y, deg_partials, bias)


# speedup vs baseline: 1.0451x; 1.0097x over previous
import functools

import jax
import jax.numpy as jnp
from jax import lax
from jax.experimental import pallas as pl
from jax.experimental.pallas import tpu as pltpu
from jax.experimental.pallas import tpu_sc as plsc

N = 10000
D = 128
E = 320000
NC = 2
NS = 16
NW = NC * NS
EPW = E // NW
WIN = 40
NWIN = EPW // WIN
AWIN = 128
APW = 10240
ANW = APW // AWIN
NPADW = APW - EPW
PADK = NW * NPADW // 16
RPW = N // NS
NPAD = 10240
DCH = NPAD // NS

_mesh = plsc.VectorSubcoreMesh(core_axis_name="c", subcore_axis_name="s")


def _deg_body(dst_hbm, ones_hbm, zeros_hbm, out_hbm, idx_v, ones_v, deg_sh, sem):
    c = lax.axis_index("c")
    s = lax.axis_index("s")
    wid = s * NC + c
    pltpu.sync_copy(zeros_hbm.at[pl.ds(s * DCH, DCH)],
                    deg_sh.at[pl.ds(s * DCH, DCH)])
    pltpu.sync_copy(dst_hbm.at[wid], idx_v)
    pltpu.sync_copy(ones_hbm, ones_v)
    plsc.subcore_barrier()

    def fire(j, carry):
        pltpu.async_copy(ones_v, deg_sh.at[idx_v.at[j]], sem, add=True)
        return carry

    def drain(j, carry):
        pltpu.make_async_copy(ones_v, deg_sh.at[idx_v.at[j]], sem).wait()
        return carry

    lax.fori_loop(0, ANW, fire, 0)
    lax.fori_loop(0, ANW, drain, 0)
    plsc.subcore_barrier()
    pltpu.sync_copy(deg_sh.at[pl.ds(s * DCH, DCH)],
                    out_hbm.at[c].at[pl.ds(s * DCH, DCH)])


_deg_kernel = functools.partial(
    pl.kernel,
    out_type=jax.ShapeDtypeStruct((NC, NPAD), jnp.float32),
    mesh=_mesh,
    scratch_types=[
        pltpu.VMEM((ANW, AWIN), jnp.int32),
        pltpu.VMEM((AWIN,), jnp.float32),
        pltpu.VMEM_SHARED((NPAD,), jnp.float32),
        pltpu.SemaphoreType.DMA,
    ],
)(_deg_body)


def _agg_body(y_hbm, src_hbm, dst_hbm, zeros_hbm, out_hbm,
              srcv, dstg, rows_a, rows_b, acc_sh,
              sem_ga, sem_gb, sem_sa, sem_sb):
    c = lax.axis_index("c")
    s = lax.axis_index("s")
    wid = s * NC + c
    pltpu.sync_copy(zeros_hbm.at[s], acc_sh.at[pl.ds(s * RPW, RPW)])
    pltpu.sync_copy(src_hbm.at[wid], srcv)
    plsc.subcore_barrier()

    rows = (rows_a, rows_b)
    sg = (sem_ga, sem_gb)
    ss = (sem_sa, sem_sb)
    NB = ANW // 8

    def g_start(w, k):
        pltpu.async_copy(y_hbm.at[srcv.at[pl.ds(w * AWIN, AWIN)]],
                         rows[k % 2], sg[k % 2])

    def g_wait(w, k):
        pltpu.make_async_copy(y_hbm.at[srcv.at[pl.ds(w * AWIN, AWIN)]],
                              rows[k % 2], sg[k % 2]).wait()

    def s_start(k):
        pltpu.async_copy(rows[k % 2], acc_sh.at[dstg.at[k % 8]],
                         ss[k % 2], add=True)

    def s_wait(k):
        pltpu.make_async_copy(rows[k % 2], acc_sh.at[dstg.at[k % 8]],
                              ss[k % 2]).wait()

    def i_load(j):
        off = pl.multiple_of(j * 8, 8)
        pltpu.sync_copy(dst_hbm.at[wid].at[pl.ds(off, 8)], dstg)

    g_start(0, 0)

    def step(j, carry):
        w0 = 8 * j

        @pl.when(j > 0)
        def _():
            s_wait(7)

        i_load(j)
        for k in range(8):
            w = w0 + k
            g_wait(w, k)
            if k > 0:
                s_wait(k - 1)
            if k == 7:
                @pl.when(j < NB - 1)
                def _():
                    g_start(w + 1, k + 1)
            else:
                g_start(w + 1, k + 1)
            s_start(k)
        return carry

    lax.fori_loop(0, NB, step, 0)
    s_wait(7)
    plsc.subcore_barrier()
    pltpu.sync_copy(acc_sh.at[pl.ds(s * RPW, RPW)], out_hbm.at[c].at[s])


_agg_kernel = functools.partial(
    pl.kernel,
    out_type=jax.ShapeDtypeStruct((NC, NS, RPW, D), jnp.float32),
    mesh=_mesh,
    scratch_types=[
        pltpu.VMEM((APW,), jnp.int32),
        pltpu.VMEM((8, AWIN), jnp.int32),
        pltpu.VMEM((AWIN, D), jnp.float32),
        pltpu.VMEM((AWIN, D), jnp.float32),
        pltpu.VMEM_SHARED((N, D), jnp.float32),
        pltpu.SemaphoreType.DMA,
        pltpu.SemaphoreType.DMA,
        pltpu.SemaphoreType.DMA,
        pltpu.SemaphoreType.DMA,
    ],
)(_agg_body)


def _scale_body(degp_ref, x_ref, w_ref, y_ref):
    row = lax.broadcasted_iota(jnp.int32, (N, 1), 0)
    padc = jnp.where(row < 16, float(PADK), 0.0)
    deg = degp_ref[0] + degp_ref[1] + 1.0 - padc
    dinv = lax.rsqrt(deg)
    xw = jnp.dot(x_ref[...], w_ref[...], preferred_element_type=jnp.float32)
    y_ref[...] = xw * dinv


def _scale(deg_partials, x, weight):
    return pl.pallas_call(
        _scale_body,
        out_shape=jax.ShapeDtypeStruct((N, D), jnp.float32),
    )(deg_partials, x, weight)


def _out_body(p_ref, y_ref, degp_ref, b_ref, o_ref):
    row = lax.broadcasted_iota(jnp.int32, (N, 1), 0)
    padc = jnp.where(row < 16, float(PADK), 0.0)
    deg = degp_ref[0] + degp_ref[1] + 1.0 - padc
    dinv = lax.rsqrt(deg)
    yfac = jnp.where(row < 16, 1.0 - PADK, 1.0)
    fr = (p_ref[0] + p_ref[1] + yfac * y_ref[...]) * dinv
    o_ref[...] = fr + b_ref[...]


def _combine(partials, y, deg_partials, bias):
    return pl.pallas_call(
        _out_body,
        out_shape=jax.ShapeDtypeStruct((N, D), jnp.float32),
    )(partials, y, deg_partials, bias.reshape(1, D))


def kernel(x, edge_index, W1, b1, W2, b2, weight, bias):
    pad = jnp.tile(jnp.arange(16, dtype=edge_index.dtype), (NW, NPADW // 16))
    srcp = jnp.concatenate([edge_index[0].reshape(NW, EPW), pad], axis=1)
    dstp = jnp.concatenate([edge_index[1].reshape(NW, EPW), pad], axis=1)
    dstp = dstp.reshape(NW, ANW, AWIN)
    ones_w = jnp.ones((AWIN,), jnp.float32)
    zeros_deg = jnp.zeros((NPAD,), jnp.float32)
    zeros_acc = jnp.zeros((NS, RPW, D), jnp.float32)

    deg_partials = _deg_kernel(dstp, ones_w, zeros_deg)[:, :N, None]
    y = _scale(deg_partials, x, weight)
    partials = _agg_kernel(y, srcp, dstp, zeros_acc).reshape(NC, N, D)
    return _combine(partials, y, deg_partials, bias)
